# baseline (speedup 1.0000x reference)
import functools

import jax
import jax.numpy as jnp
from jax import lax
from jax.experimental import pallas as pl
from jax.experimental.pallas import tpu as pltpu
from jax.experimental.pallas import tpu_sc as plsc

_VOCAB = 1_000_000
_EMB = 32
_LS = 33
_B = 16384
_F = 26
_N = _B * _F
_NC = 2
_NS = 16
_NW = _NC * _NS

_TW = 512
_NBLK = _VOCAB // _TW
_TAIL = _VOCAB - _NBLK * _TW
_K1_PAIRS = 31

_NPW = _N // _NW
_CHUNK = 128
_NCHUNK = _NPW // _CHUNK
_NBUF = 4
_BBLKS = _B // _CHUNK

_mesh = plsc.VectorSubcoreMesh(
    core_axis_name="c", subcore_axis_name="s", num_cores=_NC, num_subcores=_NS
)


def _k1_body(tt_hbm, tail_hbm, ltab_hbm, a0, a1, b0, b1, *sems):
    avs = (a0, a1)
    bvs = (b0, b1)
    rsems = sems[:2]
    wsems = sems[2:]
    wid = lax.axis_index("s") * _NC + lax.axis_index("c")
    iota = lax.iota(jnp.int32, 16)
    ilane = iota * _LS

    @pl.when(wid == 0)
    def _():
        pltpu.sync_copy(tail_hbm, b0.at[pl.ds(0, _TAIL * _EMB)])
        for v in range(_TAIL * _EMB // 16):
            p = iota + v * 16
            dst = p + lax.shift_right_logical(p, 5)
            plsc.store_scatter(b1, [dst], b0[pl.ds(v * 16, 16)])
        pltpu.sync_copy(
            b1.at[pl.ds(0, _TAIL * _LS)],
            ltab_hbm.at[pl.ds(_NBLK * _TW * _LS, _TAIL * _LS)],
        )

    def start_reads(c, buf):
        for j8 in range(4):
            pltpu.async_copy(
                tt_hbm.at[pl.ds(j8 * 8, 8), pl.ds(c * _TW, _TW)],
                avs[buf].at[pl.ds(j8 * 8, 8), :],
                rsems[buf],
            )

    def wait_reads(c, buf):
        for j8 in range(4):
            pltpu.make_async_copy(
                tt_hbm.at[pl.ds(j8 * 8, 8), pl.ds(c * _TW, _TW)],
                avs[buf].at[pl.ds(j8 * 8, 8), :],
                rsems[buf],
            ).wait()

    def transpose(buf):
        @pl.loop(0, _TW // 128)
        def _(cb):
            coff = cb * 128
            for j in range(_EMB):
                vals = [
                    avs[buf][j, pl.ds(coff + t * 16, 16)] for t in range(8)
                ]
                for t in range(8):
                    plsc.store_scatter(
                        bvs[buf],
                        [ilane + ((coff + t * 16) * _LS + j)],
                        vals[t],
                    )

    def start_write(c, buf):
        pltpu.async_copy(
            bvs[buf],
            ltab_hbm.at[pl.ds(c * (_TW * _LS), _TW * _LS)],
            wsems[buf],
        )

    def wait_write(c, buf):
        pltpu.make_async_copy(
            bvs[buf],
            ltab_hbm.at[pl.ds(c * (_TW * _LS), _TW * _LS)],
            wsems[buf],
        ).wait()

    start_reads(wid, 0)

    @pl.loop(0, _K1_PAIRS)
    def _(p):
        for buf in range(2):
            k = p * 2 + buf
            c = wid + k * _NW
            cn = c + _NW

            @pl.when(c < _NBLK)
            def _():
                @pl.when(cn < _NBLK)
                def _():
                    start_reads(cn, 1 - buf)

                wait_reads(c, buf)

                @pl.when(k >= 2)
                def _():
                    wait_write(c, buf)

                transpose(buf)
                start_write(c, buf)

    wait_write(0, 0)
    wait_write(0, 1)


_k1 = functools.partial(
    pl.kernel,
    out_type=jax.ShapeDtypeStruct((_VOCAB * _LS,), jnp.float32),
    mesh=_mesh,
    scratch_types=[
        pltpu.VMEM((_EMB, _TW), jnp.float32),
        pltpu.VMEM((_EMB, _TW), jnp.float32),
        pltpu.VMEM((_TW * _LS,), jnp.float32),
        pltpu.VMEM((_TW * _LS,), jnp.float32),
    ]
    + [pltpu.SemaphoreType.DMA] * 4,
    compiler_params=pltpu.CompilerParams(
        use_tc_tiling_on_sc=True, needs_layout_passes=False
    ),
)(_k1_body)


def _k2_body(xw_hbm, ltab_hbm, out_hbm, idx_v, a0, a1, a2, a3, b0, b1, b2, b3, *sems):
    avs = (a0, a1, a2, a3)
    bvs = (b0, b1, b2, b3)
    gsems = sems[:_NBUF]
    ssems = sems[_NBUF:]
    wid = lax.axis_index("s") * _NC + lax.axis_index("c")
    ilane = lax.iota(jnp.int32, 16)

    pltpu.sync_copy(xw_hbm.at[wid], idx_v)

    def start_gather(g, buf):
        pltpu.async_copy(
            ltab_hbm.at[idx_v.at[g]], avs[buf], gsems[buf]
        )

    def wait_gather(g, buf):
        pltpu.make_async_copy(
            ltab_hbm.at[idx_v.at[g]], avs[buf], gsems[buf]
        ).wait()

    rows_t = [ilane + t * 16 for t in range(_CHUNK // 16)]

    def transpose(buf):
        for j in range(_EMB):
            cj = jnp.full((16,), j, jnp.int32)
            vals = [
                plsc.load_gather(avs[buf], [rows_t[t], cj])
                for t in range(_CHUNK // 16)
            ]
            for t in range(_CHUNK // 16):
                bvs[buf][j, pl.ds(t * 16, 16)] = vals[t]

    def _fb(g):
        t = wid * _NCHUNK + g
        return lax.div(t, _BBLKS), lax.rem(t, _BBLKS)

    def start_stores(g, buf):
        f, bblk = _fb(g)
        pltpu.async_copy(
            bvs[buf],
            out_hbm.at[f, :, pl.ds(bblk * _CHUNK, _CHUNK)],
            ssems[buf],
        )

    def wait_stores(g, buf):
        f, bblk = _fb(g)
        pltpu.make_async_copy(
            bvs[buf],
            out_hbm.at[f, :, pl.ds(bblk * _CHUNK, _CHUNK)],
            ssems[buf],
        ).wait()

    for buf in range(_NBUF):
        start_gather(buf, buf)

    @pl.loop(0, _NCHUNK // _NBUF)
    def _(p):
        for buf in range(_NBUF):
            g = p * _NBUF + buf
            wait_gather(g, buf)

            @pl.when(g >= _NBUF)
            def _():
                wait_stores(g - _NBUF, buf)

            transpose(buf)
            start_stores(g, buf)

            @pl.when(g + _NBUF < _NCHUNK)
            def _():
                start_gather(g + _NBUF, buf)

    for buf in range(_NBUF):
        wait_stores(_NCHUNK - _NBUF + buf, buf)


_k2 = functools.partial(
    pl.kernel,
    out_type=jax.ShapeDtypeStruct((_F, _EMB, _B), jnp.float32),
    mesh=_mesh,
    scratch_types=[
        pltpu.VMEM((_NCHUNK, _CHUNK), jnp.int32),
        pltpu.VMEM((_CHUNK, _LS), jnp.float32),
        pltpu.VMEM((_CHUNK, _LS), jnp.float32),
        pltpu.VMEM((_CHUNK, _LS), jnp.float32),
        pltpu.VMEM((_CHUNK, _LS), jnp.float32),
        pltpu.VMEM((_EMB, _CHUNK), jnp.float32),
        pltpu.VMEM((_EMB, _CHUNK), jnp.float32),
        pltpu.VMEM((_EMB, _CHUNK), jnp.float32),
        pltpu.VMEM((_EMB, _CHUNK), jnp.float32),
    ]
    + [pltpu.SemaphoreType.DMA] * (2 * _NBUF),
    compiler_params=pltpu.CompilerParams(
        use_tc_tiling_on_sc=False, needs_layout_passes=False
    ),
)(_k2_body)


@jax.jit
def kernel(x, table):
    xw = jnp.transpose(x).reshape(_NW, _NCHUNK, _CHUNK)
    tt = jnp.transpose(table)
    tail = table[_NBLK * _TW :].reshape(-1)
    ltab = _k1(tt, tail)
    o3 = _k2(xw, ltab.reshape(_VOCAB, _LS))
    return jnp.transpose(o3, (2, 0, 1))


# speedup vs baseline: 4.5084x; 4.5084x over previous
import functools

import jax
import jax.numpy as jnp
from jax import lax
from jax.experimental import pallas as pl
from jax.experimental.pallas import tpu as pltpu
from jax.experimental.pallas import tpu_sc as plsc

_VOCAB = 1_000_000
_EMB = 32
_LS = 40
_B = 16384
_F = 26
_N = _B * _F
_NC = 2
_NS = 16
_NW = _NC * _NS

_TW = 512
_NBLK = _VOCAB // _TW
_TAIL = _VOCAB - _NBLK * _TW
_K1_PAIRS = 31

_NPW = _N // _NW
_CHUNK = 128
_NCHUNK = _NPW // _CHUNK
_NBUF = 4
_BBLKS = _B // _CHUNK

_mesh = plsc.VectorSubcoreMesh(
    core_axis_name="c", subcore_axis_name="s", num_cores=_NC, num_subcores=_NS
)


def _k1_body(tt_hbm, tail_hbm, ltab_hbm, a0, a1, b0, b1, *sems):
    avs = (a0, a1)
    bvs = (b0, b1)
    rsems = sems[:2]
    wsems = sems[2:]
    wid = lax.axis_index("s") * _NC + lax.axis_index("c")
    iota = lax.iota(jnp.int32, 16)
    ilane = iota * _LS

    @pl.when(wid == 0)
    def _():
        pltpu.sync_copy(tail_hbm, b0.at[pl.ds(0, _TAIL * _EMB)])
        for v in range(_TAIL * _EMB // 16):
            p = iota + v * 16
            dst = lax.shift_right_logical(p, 5) * _LS + (p & 31)
            plsc.store_scatter(b1, [dst], b0[pl.ds(v * 16, 16)])
        pltpu.sync_copy(
            b1.at[pl.ds(0, _TAIL * _LS)],
            ltab_hbm.at[pl.ds(_NBLK * _TW * _LS, _TAIL * _LS)],
        )

    def start_reads(c, buf):
        for j8 in range(4):
            pltpu.async_copy(
                tt_hbm.at[pl.ds(j8 * 8, 8), pl.ds(c * _TW, _TW)],
                avs[buf].at[pl.ds(j8 * 8, 8), :],
                rsems[buf],
            )

    def wait_reads(c, buf):
        for j8 in range(4):
            pltpu.make_async_copy(
                tt_hbm.at[pl.ds(j8 * 8, 8), pl.ds(c * _TW, _TW)],
                avs[buf].at[pl.ds(j8 * 8, 8), :],
                rsems[buf],
            ).wait()

    def transpose(buf):
        @pl.loop(0, _TW // 128)
        def _(cb):
            coff = cb * 128
            for j in range(_EMB):
                vals = [
                    avs[buf][j, pl.ds(coff + t * 16, 16)] for t in range(8)
                ]
                for t in range(8):
                    plsc.store_scatter(
                        bvs[buf],
                        [ilane + ((coff + t * 16) * _LS + j)],
                        vals[t],
                    )

    def start_write(c, buf):
        pltpu.async_copy(
            bvs[buf],
            ltab_hbm.at[pl.ds(c * (_TW * _LS), _TW * _LS)],
            wsems[buf],
        )

    def wait_write(c, buf):
        pltpu.make_async_copy(
            bvs[buf],
            ltab_hbm.at[pl.ds(c * (_TW * _LS), _TW * _LS)],
            wsems[buf],
        ).wait()

    start_reads(wid, 0)

    @pl.loop(0, _K1_PAIRS)
    def _(p):
        for buf in range(2):
            k = p * 2 + buf
            c = wid + k * _NW
            cn = c + _NW

            @pl.when(c < _NBLK)
            def _():
                @pl.when(cn < _NBLK)
                def _():
                    start_reads(cn, 1 - buf)

                wait_reads(c, buf)

                @pl.when(k >= 2)
                def _():
                    wait_write(c, buf)

                transpose(buf)
                start_write(c, buf)

    wait_write(0, 0)
    wait_write(0, 1)


_k1 = functools.partial(
    pl.kernel,
    out_type=jax.ShapeDtypeStruct((_VOCAB * _LS,), jnp.float32),
    mesh=_mesh,
    scratch_types=[
        pltpu.VMEM((_EMB, _TW), jnp.float32),
        pltpu.VMEM((_EMB, _TW), jnp.float32),
        pltpu.VMEM((_TW * _LS,), jnp.float32),
        pltpu.VMEM((_TW * _LS,), jnp.float32),
    ]
    + [pltpu.SemaphoreType.DMA] * 4,
    compiler_params=pltpu.CompilerParams(
        use_tc_tiling_on_sc=True, needs_layout_passes=False
    ),
)(_k1_body)


def _k2_body(xw_hbm, ltab_hbm, out_hbm, idx_v, a0, a1, a2, a3, b0, b1, b2, b3, *sems):
    avs = (a0, a1, a2, a3)
    bvs = (b0, b1, b2, b3)
    gsems = sems[:_NBUF]
    ssems = sems[_NBUF:]
    wid = lax.axis_index("s") * _NC + lax.axis_index("c")
    ilane = lax.iota(jnp.int32, 16)

    pltpu.sync_copy(xw_hbm.at[wid], idx_v)

    def start_gather(g, buf):
        pltpu.async_copy(
            ltab_hbm.at[idx_v.at[g]], avs[buf], gsems[buf]
        )

    def wait_gather(g, buf):
        pltpu.make_async_copy(
            ltab_hbm.at[idx_v.at[g]], avs[buf], gsems[buf]
        ).wait()

    rows_t = [ilane + t * 16 for t in range(_CHUNK // 16)]

    def transpose(buf):
        for j in range(_EMB):
            cj = jnp.full((16,), j, jnp.int32)
            vals = [
                plsc.load_gather(avs[buf], [rows_t[t], cj])
                for t in range(_CHUNK // 16)
            ]
            for t in range(_CHUNK // 16):
                bvs[buf][j, pl.ds(t * 16, 16)] = vals[t]

    def _fb(g):
        t = wid * _NCHUNK + g
        return lax.div(t, _BBLKS), lax.rem(t, _BBLKS)

    def start_stores(g, buf):
        f, bblk = _fb(g)
        pltpu.async_copy(
            bvs[buf],
            out_hbm.at[f, :, pl.ds(bblk * _CHUNK, _CHUNK)],
            ssems[buf],
        )

    def wait_stores(g, buf):
        f, bblk = _fb(g)
        pltpu.make_async_copy(
            bvs[buf],
            out_hbm.at[f, :, pl.ds(bblk * _CHUNK, _CHUNK)],
            ssems[buf],
        ).wait()

    for buf in range(_NBUF):
        start_gather(buf, buf)

    @pl.loop(0, _NCHUNK // _NBUF)
    def _(p):
        for buf in range(_NBUF):
            g = p * _NBUF + buf
            wait_gather(g, buf)

            @pl.when(g >= _NBUF)
            def _():
                wait_stores(g - _NBUF, buf)

            transpose(buf)
            start_stores(g, buf)

            @pl.when(g + _NBUF < _NCHUNK)
            def _():
                start_gather(g + _NBUF, buf)

    for buf in range(_NBUF):
        wait_stores(_NCHUNK - _NBUF + buf, buf)


_k2 = functools.partial(
    pl.kernel,
    out_type=jax.ShapeDtypeStruct((_F, _EMB, _B), jnp.float32),
    mesh=_mesh,
    scratch_types=[
        pltpu.VMEM((_NCHUNK, _CHUNK), jnp.int32),
        pltpu.VMEM((_CHUNK, _LS), jnp.float32),
        pltpu.VMEM((_CHUNK, _LS), jnp.float32),
        pltpu.VMEM((_CHUNK, _LS), jnp.float32),
        pltpu.VMEM((_CHUNK, _LS), jnp.float32),
        pltpu.VMEM((_EMB, _CHUNK), jnp.float32),
        pltpu.VMEM((_EMB, _CHUNK), jnp.float32),
        pltpu.VMEM((_EMB, _CHUNK), jnp.float32),
        pltpu.VMEM((_EMB, _CHUNK), jnp.float32),
    ]
    + [pltpu.SemaphoreType.DMA] * (2 * _NBUF),
    compiler_params=pltpu.CompilerParams(
        use_tc_tiling_on_sc=False, needs_layout_passes=False
    ),
)(_k2_body)


@jax.jit
def kernel(x, table):
    xw = jnp.transpose(x).reshape(_NW, _NCHUNK, _CHUNK)
    tt = jnp.transpose(table)
    tail = table[_NBLK * _TW :].reshape(-1)
    ltab = _k1(tt, tail)
    o3 = _k2(xw, ltab.reshape(_VOCAB, _LS))
    return jnp.transpose(o3, (2, 0, 1))


# speedup vs baseline: 4.5661x; 1.0128x over previous
import functools

import jax
import jax.numpy as jnp
from jax import lax
from jax.experimental import pallas as pl
from jax.experimental.pallas import tpu as pltpu
from jax.experimental.pallas import tpu_sc as plsc

_VOCAB = 1_000_000
_EMB = 32
_LS = 40
_B = 16384
_F = 26
_N = _B * _F
_NC = 2
_NS = 16
_NW = _NC * _NS

_TW = 512
_NBLK = _VOCAB // _TW
_TAIL = _VOCAB - _NBLK * _TW
_K1_PAIRS = 31

_NPW = _N // _NW
_CHUNK = 128
_NCHUNK = _NPW // _CHUNK
_NBUF = 4
_BBLKS = _B // _CHUNK

_mesh = plsc.VectorSubcoreMesh(
    core_axis_name="c", subcore_axis_name="s", num_cores=_NC, num_subcores=_NS
)


def _k1_body(tt_hbm, tail_hbm, ltab_hbm, a0, a1, b0, b1, *sems):
    avs = (a0, a1)
    bvs = (b0, b1)
    rsems = sems[:2]
    wsems = sems[2:]
    wid = lax.axis_index("s") * _NC + lax.axis_index("c")
    iota = lax.iota(jnp.int32, 16)
    ilane = iota * _LS

    @pl.when(wid == 0)
    def _():
        pltpu.sync_copy(tail_hbm, b0.at[pl.ds(0, _TAIL * _EMB)])
        for v in range(_TAIL * _EMB // 16):
            p = iota + v * 16
            dst = lax.shift_right_logical(p, 5) * _LS + (p & 31)
            plsc.store_scatter(b1, [dst], b0[pl.ds(v * 16, 16)])
        pltpu.sync_copy(
            b1.at[pl.ds(0, _TAIL * _LS)],
            ltab_hbm.at[pl.ds(_NBLK * _TW * _LS, _TAIL * _LS)],
        )

    def start_reads(c, buf):
        for j8 in range(4):
            pltpu.async_copy(
                tt_hbm.at[pl.ds(j8 * 8, 8), pl.ds(c * _TW, _TW)],
                avs[buf].at[pl.ds(j8 * 8, 8), :],
                rsems[buf],
            )

    def wait_reads(c, buf):
        for j8 in range(4):
            pltpu.make_async_copy(
                tt_hbm.at[pl.ds(j8 * 8, 8), pl.ds(c * _TW, _TW)],
                avs[buf].at[pl.ds(j8 * 8, 8), :],
                rsems[buf],
            ).wait()

    def transpose(buf):
        @pl.loop(0, _TW // 128)
        def _(cb):
            coff = cb * 128

            def loads(j):
                return [
                    avs[buf][j, pl.ds(coff + t * 16, 16)] for t in range(8)
                ]

            def stores(j, vals):
                for t in range(8):
                    plsc.store_scatter(
                        bvs[buf],
                        [ilane + ((coff + t * 16) * _LS + j)],
                        vals[t],
                    )

            vals = loads(0)
            for j in range(1, _EMB):
                nxt = loads(j)
                stores(j - 1, vals)
                vals = nxt
            stores(_EMB - 1, vals)

    def start_write(c, buf):
        pltpu.async_copy(
            bvs[buf],
            ltab_hbm.at[pl.ds(c * (_TW * _LS), _TW * _LS)],
            wsems[buf],
        )

    def wait_write(c, buf):
        pltpu.make_async_copy(
            bvs[buf],
            ltab_hbm.at[pl.ds(c * (_TW * _LS), _TW * _LS)],
            wsems[buf],
        ).wait()

    start_reads(wid, 0)

    @pl.loop(0, _K1_PAIRS)
    def _(p):
        for buf in range(2):
            k = p * 2 + buf
            c = wid + k * _NW
            cn = c + _NW

            @pl.when(c < _NBLK)
            def _():
                @pl.when(cn < _NBLK)
                def _():
                    start_reads(cn, 1 - buf)

                wait_reads(c, buf)

                @pl.when(k >= 2)
                def _():
                    wait_write(c, buf)

                transpose(buf)
                start_write(c, buf)

    wait_write(0, 0)
    wait_write(0, 1)


_k1 = functools.partial(
    pl.kernel,
    out_type=jax.ShapeDtypeStruct((_VOCAB * _LS,), jnp.float32),
    mesh=_mesh,
    scratch_types=[
        pltpu.VMEM((_EMB, _TW), jnp.float32),
        pltpu.VMEM((_EMB, _TW), jnp.float32),
        pltpu.VMEM((_TW * _LS,), jnp.float32),
        pltpu.VMEM((_TW * _LS,), jnp.float32),
    ]
    + [pltpu.SemaphoreType.DMA] * 4,
    compiler_params=pltpu.CompilerParams(
        use_tc_tiling_on_sc=True, needs_layout_passes=False
    ),
)(_k1_body)


def _k2_body(xw_hbm, ltab_hbm, out_hbm, idx_v, a0, a1, a2, a3, b0, b1, b2, b3, *sems):
    avs = (a0, a1, a2, a3)
    bvs = (b0, b1, b2, b3)
    gsems = sems[:_NBUF]
    ssems = sems[_NBUF:]
    wid = lax.axis_index("s") * _NC + lax.axis_index("c")
    ilane = lax.iota(jnp.int32, 16)

    pltpu.sync_copy(xw_hbm.at[wid], idx_v)

    def start_gather(g, buf):
        pltpu.async_copy(
            ltab_hbm.at[idx_v.at[g]], avs[buf], gsems[buf]
        )

    def wait_gather(g, buf):
        pltpu.make_async_copy(
            ltab_hbm.at[idx_v.at[g]], avs[buf], gsems[buf]
        ).wait()

    rows_t = [ilane + t * 16 for t in range(_CHUNK // 16)]

    def transpose(buf):
        def gathers(j):
            cj = jnp.full((16,), j, jnp.int32)
            return [
                plsc.load_gather(avs[buf], [rows_t[t], cj])
                for t in range(_CHUNK // 16)
            ]

        def stores(j, vals):
            for t in range(_CHUNK // 16):
                bvs[buf][j, pl.ds(t * 16, 16)] = vals[t]

        vals = gathers(0)
        for j in range(1, _EMB):
            nxt = gathers(j)
            stores(j - 1, vals)
            vals = nxt
        stores(_EMB - 1, vals)

    def _fb(g):
        t = wid * _NCHUNK + g
        return lax.div(t, _BBLKS), lax.rem(t, _BBLKS)

    def start_stores(g, buf):
        f, bblk = _fb(g)
        pltpu.async_copy(
            bvs[buf],
            out_hbm.at[f, :, pl.ds(bblk * _CHUNK, _CHUNK)],
            ssems[buf],
        )

    def wait_stores(g, buf):
        f, bblk = _fb(g)
        pltpu.make_async_copy(
            bvs[buf],
            out_hbm.at[f, :, pl.ds(bblk * _CHUNK, _CHUNK)],
            ssems[buf],
        ).wait()

    for buf in range(_NBUF):
        start_gather(buf, buf)

    @pl.loop(0, _NCHUNK // _NBUF)
    def _(p):
        for buf in range(_NBUF):
            g = p * _NBUF + buf
            wait_gather(g, buf)

            @pl.when(g >= _NBUF)
            def _():
                wait_stores(g - _NBUF, buf)

            transpose(buf)
            start_stores(g, buf)

            @pl.when(g + _NBUF < _NCHUNK)
            def _():
                start_gather(g + _NBUF, buf)

    for buf in range(_NBUF):
        wait_stores(_NCHUNK - _NBUF + buf, buf)


_k2 = functools.partial(
    pl.kernel,
    out_type=jax.ShapeDtypeStruct((_F, _EMB, _B), jnp.float32),
    mesh=_mesh,
    scratch_types=[
        pltpu.VMEM((_NCHUNK, _CHUNK), jnp.int32),
        pltpu.VMEM((_CHUNK, _LS), jnp.float32),
        pltpu.VMEM((_CHUNK, _LS), jnp.float32),
        pltpu.VMEM((_CHUNK, _LS), jnp.float32),
        pltpu.VMEM((_CHUNK, _LS), jnp.float32),
        pltpu.VMEM((_EMB, _CHUNK), jnp.float32),
        pltpu.VMEM((_EMB, _CHUNK), jnp.float32),
        pltpu.VMEM((_EMB, _CHUNK), jnp.float32),
        pltpu.VMEM((_EMB, _CHUNK), jnp.float32),
    ]
    + [pltpu.SemaphoreType.DMA] * (2 * _NBUF),
    compiler_params=pltpu.CompilerParams(
        use_tc_tiling_on_sc=False, needs_layout_passes=False
    ),
)(_k2_body)


@jax.jit
def kernel(x, table):
    xw = jnp.transpose(x).reshape(_NW, _NCHUNK, _CHUNK)
    tt = jnp.transpose(table)
    tail = table[_NBLK * _TW :].reshape(-1)
    ltab = _k1(tt, tail)
    o3 = _k2(xw, ltab.reshape(_VOCAB, _LS))
    return jnp.transpose(o3, (2, 0, 1))
